# Initial kernel scaffold; baseline (speedup 1.0000x reference)
import jax
import jax.numpy as jnp
from jax.experimental import pallas as pl


def kernel(images, stem_w, stem_s, stem_b, b1_exp_w, b1_exp_s, b1_exp_b, b1_dw_w, b1_dw_s, b1_dw_b, b1_proj_w, b1_proj_s, b1_proj_b, b2_exp_w, b2_exp_s, b2_exp_b, b2_dw_w, b2_dw_s, b2_dw_b, b2_proj_w, b2_proj_s, b2_proj_b, b3_exp_w, b3_exp_s, b3_exp_b, b3_dw_w, b3_dw_s, b3_dw_b, b3_proj_w, b3_proj_s, b3_proj_b, head_w, head_s, head_b):
    raise NotImplementedError("write your pallas kernel here")


# speedup vs baseline: 2.0114x; 2.0114x over previous
import functools

import jax
import jax.numpy as jnp
from jax.experimental import pallas as pl
from jax.experimental.pallas import tpu as pltpu


C = 128
H1 = 64
W1 = 64
WH = W1 // 2
M1 = H1 * W1
H2 = H1 // 2
M2 = H2 * WH
KP = 32


def _dw_pair(hE, hO, dww, base):
    accE = (hO[:, 0:WH, :] * dww[base + 0]
            + hE[:, 0:WH, :] * dww[base + 1]
            + hO[:, 1:WH + 1, :] * dww[base + 2])
    accO = (hE[:, 0:WH, :] * dww[base + 0]
            + hO[:, 1:WH + 1, :] * dww[base + 1]
            + hE[:, 1:WH + 1, :] * dww[base + 2])
    return accE, accO


def _body(p_ref, sw, ss, sb,
          e1w, e1s, e1b, d1w, d1s, d1b, p1w, p1s, p1b,
          e2w, e2s, e2b, d2w, d2s, d2b, p2w, p2s, p2b,
          e3w, e3s, e3b, d3w, d3s, d3b, p3w, p3s, p3b,
          hw, hs, hb,
          o_ref,
          hE, hO, hp3):
    f32 = jnp.float32
    bf16 = jnp.bfloat16

    zr = jnp.zeros((1, WH + 1, C), f32)
    zc = jnp.zeros((H1 + 2, 1, C), f32)
    hE[0:1, :, :] = zr
    hE[H1 + 1:H1 + 2, :, :] = zr
    hE[:, WH:WH + 1, :] = zc
    hO[0:1, :, :] = zr
    hO[H1 + 1:H1 + 2, :, :] = zr
    hO[:, 0:1, :] = zc
    z3 = jnp.zeros((1, H2 + 2, C), f32)
    hp3[0:1, :, :] = z3
    hp3[H2 + 1:H2 + 2, :, :] = z3
    hp3[:, 0:1, :] = jnp.zeros((H2 + 2, 1, C), f32)
    hp3[:, H2 + 1:H2 + 2, :] = jnp.zeros((H2 + 2, 1, C), f32)

    s = jnp.dot(p_ref[...], sw[...], preferred_element_type=f32)
    s = jnp.clip(s * ss[...] + sb[...], 0.0, 6.0)
    stem = s.astype(bf16)

    h = jnp.dot(stem, e1w[...], preferred_element_type=f32)
    h = jnp.clip(h * e1s[...] + e1b[...], 0.0, 6.0)
    hE[1:H1 + 1, 0:WH, :] = h[0:M1 // 2].reshape(H1, WH, C)
    hO[1:H1 + 1, 1:WH + 1, :] = h[M1 // 2:M1].reshape(H1, WH, C)

    dww = d1w[...]
    accE = jnp.zeros((H1, WH, C), f32)
    accO = jnp.zeros((H1, WH, C), f32)
    for dy in range(3):
        eS = hE[dy:dy + H1, :, :]
        oS = hO[dy:dy + H1, :, :]
        aE, aO = _dw_pair(eS, oS, dww, 3 * dy)
        accE += aE
        accO += aO
    accE = jnp.clip(accE * d1s[...] + d1b[...], 0.0, 6.0)
    accO = jnp.clip(accO * d1s[...] + d1b[...], 0.0, 6.0)
    dw1 = jnp.concatenate(
        [accE.reshape(M1 // 2, C), accO.reshape(M1 // 2, C)], axis=0
    ).astype(bf16)

    y = jnp.dot(dw1, p1w[...], preferred_element_type=f32)
    y = (y * p1s[...] + p1b[...] + stem.astype(f32)).astype(bf16)

    h2 = jnp.dot(y, e2w[...], preferred_element_type=f32)
    h2 = jnp.clip(h2 * e2s[...] + e2b[...], 0.0, 6.0)
    hE[1:H1 + 1, 0:WH, :] = h2[0:M1 // 2].reshape(H1, WH, C)
    hO[1:H1 + 1, 1:WH + 1, :] = h2[M1 // 2:M1].reshape(H1, WH, C)

    eV = hE[...].reshape((H1 + 2) // 2, 2, WH + 1, C)
    oV = hO[...].reshape((H1 + 2) // 2, 2, WH + 1, C)
    rowsE = (eV[:, 0][0:H2], eV[:, 1][0:H2], eV[:, 0][1:H2 + 1])
    rowsO = (oV[:, 0][0:H2], oV[:, 1][0:H2], oV[:, 0][1:H2 + 1])
    dww2 = d2w[...]
    acc2 = jnp.zeros((H2, WH, C), f32)
    for dy in range(3):
        rE, rO = rowsE[dy], rowsO[dy]
        acc2 += (rO[:, 0:WH, :] * dww2[3 * dy + 0]
                 + rE[:, 0:WH, :] * dww2[3 * dy + 1]
                 + rO[:, 1:WH + 1, :] * dww2[3 * dy + 2])
    acc2 = jnp.clip(acc2 * d2s[...] + d2b[...], 0.0, 6.0)
    dw2 = acc2.reshape(M2, C).astype(bf16)

    b2 = jnp.dot(dw2, p2w[...], preferred_element_type=f32)
    b2 = (b2 * p2s[...] + p2b[...]).astype(bf16)

    h3 = jnp.dot(b2, e3w[...], preferred_element_type=f32)
    h3 = jnp.clip(h3 * e3s[...] + e3b[...], 0.0, 6.0)
    hp3[1:H2 + 1, 1:H2 + 1, :] = h3.reshape(H2, H2, C)

    dww3 = d3w[...]
    acc3 = jnp.zeros((H2, H2, C), f32)
    for dy in range(3):
        row = hp3[dy:dy + H2, :, :]
        for dx in range(3):
            acc3 += row[:, dx:dx + H2, :] * dww3[3 * dy + dx]
    acc3 = jnp.clip(acc3 * d3s[...] + d3b[...], 0.0, 6.0)
    dw3 = acc3.reshape(M2, C).astype(bf16)

    b3 = jnp.dot(dw3, p3w[...], preferred_element_type=f32)
    b3 = b3 * p3s[...] + p3b[...] + b2.astype(f32)

    hact = jnp.dot(b3.astype(bf16), hw[...], preferred_element_type=f32)
    hact = jnp.clip(hact * hs[...] + hb[...], 0.0, 6.0)
    o_ref[...] = (jnp.sum(hact, axis=0, keepdims=True)
                  * (1.0 / M2)).astype(o_ref.dtype)


def _build_patches(images):
    n = images.shape[0]
    x = jnp.transpose(images, (0, 2, 3, 1))
    xp = jnp.pad(x, ((0, 0), (1, 1), (1, 1), (0, 0)))
    cols = [xp[:, dy:dy + 2 * H1:2, dx:dx + 2 * W1:2, :]
            for dy in range(3) for dx in range(3)]
    patches = jnp.stack(cols, axis=3).reshape(n, H1, W1, 27)
    pe = patches[:, :, 0::2, :].reshape(n, H1 * WH, 27)
    po = patches[:, :, 1::2, :].reshape(n, H1 * WH, 27)
    pcat = jnp.concatenate([pe, po], axis=1)
    return jnp.pad(pcat, ((0, 0), (0, 0), (0, KP - 27)))


def kernel(images, stem_w, stem_s, stem_b,
           b1_exp_w, b1_exp_s, b1_exp_b, b1_dw_w, b1_dw_s, b1_dw_b,
           b1_proj_w, b1_proj_s, b1_proj_b,
           b2_exp_w, b2_exp_s, b2_exp_b, b2_dw_w, b2_dw_s, b2_dw_b,
           b2_proj_w, b2_proj_s, b2_proj_b,
           b3_exp_w, b3_exp_s, b3_exp_b, b3_dw_w, b3_dw_s, b3_dw_b,
           b3_proj_w, b3_proj_s, b3_proj_b,
           head_w, head_s, head_b):
    n = images.shape[0]
    pcat = _build_patches(images)
    sw = stem_w[:KP, :]

    full = lambda i: (0, 0)
    wspec = lambda r: pl.BlockSpec((r, C), full)
    bspecs = [wspec(C), wspec(1), wspec(1),
              wspec(9), wspec(1), wspec(1),
              wspec(C), wspec(1), wspec(1)]

    out = pl.pallas_call(
        _body,
        grid=(n,),
        in_specs=([pl.BlockSpec((None, M1, KP), lambda i: (i, 0, 0)),
                   wspec(KP), wspec(1), wspec(1)]
                  + bspecs + bspecs + bspecs
                  + [wspec(C), wspec(1), wspec(1)]),
        out_specs=pl.BlockSpec((None, 1, C), lambda i: (i, 0, 0)),
        out_shape=jax.ShapeDtypeStruct((n, 1, C), jnp.bfloat16),
        scratch_shapes=[
            pltpu.VMEM((H1 + 2, WH + 1, C), jnp.float32),
            pltpu.VMEM((H1 + 2, WH + 1, C), jnp.float32),
            pltpu.VMEM((H2 + 2, H2 + 2, C), jnp.float32),
        ],
        compiler_params=pltpu.CompilerParams(
            dimension_semantics=("parallel",)),
    )(pcat, sw, stem_s, stem_b,
      b1_exp_w, b1_exp_s, b1_exp_b, b1_dw_w, b1_dw_s, b1_dw_b,
      b1_proj_w, b1_proj_s, b1_proj_b,
      b2_exp_w, b2_exp_s, b2_exp_b, b2_dw_w, b2_dw_s, b2_dw_b,
      b2_proj_w, b2_proj_s, b2_proj_b,
      b3_exp_w, b3_exp_s, b3_exp_b, b3_dw_w, b3_dw_s, b3_dw_b,
      b3_proj_w, b3_proj_s, b3_proj_b,
      head_w, head_s, head_b)
    return out.astype(images.dtype)


# speedup vs baseline: 2.1530x; 1.0704x over previous
import jax
import jax.numpy as jnp
from jax.experimental import pallas as pl
from jax.experimental.pallas import tpu as pltpu


C = 128
H1 = 64
W1 = 64
WH = W1 // 2
M1 = H1 * W1
MH = M1 // 2
H2 = H1 // 2
M2 = H2 * WH
KP = 32
CE = 64
C3 = 96


def _body(p_ref, sw, ss, sb,
          e1w, e1sp, e1bp, d1wp, d1sp, d1bp, p1lo, p1hi, p1s, p1b,
          e2w, e2sp, e2bp, d2wp, d2sp, d2bp, p2z, p2s, p2b,
          e3w, e3s, e3b, d3w, d3s, d3b, p3w, p3s, p3b,
          hw, hs, hb,
          o_ref,
          hP, hR, hp3):
    f32 = jnp.float32
    bf16 = jnp.bfloat16
    mask = jax.lax.broadcasted_iota(jnp.int32, (H1, WH, C), 2) < CE

    hP[0:1, :, :] = jnp.zeros((1, WH, C), f32)
    hP[H1 + 1:H1 + 2, :, :] = jnp.zeros((1, WH, C), f32)
    hR[0:1, :, :] = jnp.zeros((1, WH + 2, C), f32)
    hR[H1 + 1:H1 + 2, :, :] = jnp.zeros((1, WH + 2, C), f32)
    hR[:, 0:1, :] = jnp.zeros((H1 + 2, 1, C), f32)
    hR[:, WH + 1:WH + 2, :] = jnp.zeros((H1 + 2, 1, C), f32)
    hp3[0:1, :, :] = jnp.zeros((1, H2 + 2, C3), f32)
    hp3[H2 + 1:H2 + 2, :, :] = jnp.zeros((1, H2 + 2, C3), f32)
    hp3[:, 0:1, :] = jnp.zeros((H2 + 2, 1, C3), f32)
    hp3[:, H2 + 1:H2 + 2, :] = jnp.zeros((H2 + 2, 1, C3), f32)

    s = jnp.dot(p_ref[...], sw[...], preferred_element_type=f32)
    s = jnp.clip(s * ss[...] + sb[...], 0.0, 6.0)
    sEb = s[0:MH].astype(bf16)
    sOb = s[MH:M1].astype(bf16)

    def packed_dw_input(hEv, hOv, scale, bias):
        P = jnp.concatenate([hEv, hOv], axis=1)
        P = jnp.clip(P * scale + bias, 0.0, 6.0)
        R = jnp.concatenate([P[:, CE:C], P[:, 0:CE]], axis=1)
        hP[1:H1 + 1, 0:WH, :] = P.reshape(H1, WH, C)
        hR[1:H1 + 1, 1:WH + 1, :] = R.reshape(H1, WH, C)

    e1 = e1w[...]
    packed_dw_input(jnp.dot(sEb, e1, preferred_element_type=f32),
                    jnp.dot(sOb, e1, preferred_element_type=f32),
                    e1sp[...], e1bp[...])
    dww = d1wp[...]
    acc = jnp.zeros((H1, WH, C), f32)
    for dy in range(3):
        rP = hP[dy:dy + H1, :, :]
        rR = hR[dy:dy + H1, :, :]
        A0 = jnp.where(mask, rR[:, 0:WH, :], rR[:, 1:WH + 1, :])
        A2 = jnp.where(mask, rR[:, 1:WH + 1, :], rR[:, 2:WH + 2, :])
        acc += (A0 * dww[3 * dy + 0] + rP * dww[3 * dy + 1]
                + A2 * dww[3 * dy + 2])
    acc = jnp.clip(acc * d1sp[...] + d1bp[...], 0.0, 6.0)
    OPb = acc.astype(bf16).reshape(MH, C)

    yE = jnp.dot(OPb, p1lo[...], preferred_element_type=f32)
    yE = ((yE * p1s[...] + p1b[...]) + sEb.astype(f32)).astype(bf16)
    yO = jnp.dot(OPb, p1hi[...], preferred_element_type=f32)
    yO = ((yO * p1s[...] + p1b[...]) + sOb.astype(f32)).astype(bf16)

    e2 = e2w[...]
    packed_dw_input(jnp.dot(yE, e2, preferred_element_type=f32),
                    jnp.dot(yO, e2, preferred_element_type=f32),
                    e2sp[...], e2bp[...])
    pV = hP[...].reshape((H1 + 2) // 2, 2, WH, C)
    rV = hR[...].reshape((H1 + 2) // 2, 2, WH + 2, C)
    rowsP = (pV[:, 0][0:H2], pV[:, 1][0:H2], pV[:, 0][1:H2 + 1])
    rowsR = (rV[:, 0][0:H2], rV[:, 1][0:H2], rV[:, 0][1:H2 + 1])
    dww2 = d2wp[...]
    acc2 = jnp.zeros((H2, WH, C), f32)
    for dy in range(3):
        rP, rR = rowsP[dy], rowsR[dy]
        acc2 += (rR[:, 0:WH, :] * dww2[3 * dy + 0]
                 + rP[:, 0:WH, :] * dww2[3 * dy + 1]
                 + rR[:, 1:WH + 1, :] * dww2[3 * dy + 2])
    acc2 = jnp.clip(acc2 * d2sp[...] + d2bp[...], 0.0, 6.0)
    d2b = acc2.astype(bf16).reshape(M2, C)

    b2 = jnp.dot(d2b, p2z[...], preferred_element_type=f32)
    b2v = (b2 * p2s[...] + p2b[...]).astype(bf16)

    h3 = jnp.dot(b2v, e3w[...], preferred_element_type=f32)
    h3 = jnp.clip(h3 * e3s[...] + e3b[...], 0.0, 6.0)
    hp3[1:H2 + 1, 1:H2 + 1, :] = h3.reshape(H2, H2, C3)

    dww3 = d3w[...]
    acc3 = jnp.zeros((H2, H2, C3), f32)
    for dy in range(3):
        row = hp3[dy:dy + H2, :, :]
        for dx in range(3):
            acc3 += row[:, dx:dx + H2, :] * dww3[3 * dy + dx]
    acc3 = jnp.clip(acc3 * d3s[...] + d3b[...], 0.0, 6.0)
    d3v = acc3.astype(bf16).reshape(M2, C3)

    b3 = jnp.dot(d3v, p3w[...], preferred_element_type=f32)
    b3 = b3 * p3s[...] + p3b[...] + b2v.astype(f32)

    hact = jnp.dot(b3.astype(bf16), hw[...], preferred_element_type=f32)
    hact = jnp.clip(hact * hs[...] + hb[...], 0.0, 6.0)
    o_ref[...] = (jnp.sum(hact, axis=0, keepdims=True)
                  * (1.0 / M2)).astype(o_ref.dtype)


def _build_patches(images):
    n = images.shape[0]
    x = jnp.transpose(images, (0, 2, 3, 1))
    xp = jnp.pad(x, ((0, 0), (1, 1), (1, 1), (0, 0)))
    cols = [xp[:, dy:dy + 2 * H1:2, dx:dx + 2 * W1:2, :]
            for dy in range(3) for dx in range(3)]
    patches = jnp.stack(cols, axis=3).reshape(n, H1, W1, 27)
    pe = patches[:, :, 0::2, :].reshape(n, H1 * WH, 27)
    po = patches[:, :, 1::2, :].reshape(n, H1 * WH, 27)
    pcat = jnp.concatenate([pe, po], axis=1)
    return jnp.pad(pcat, ((0, 0), (0, 0), (0, KP - 27)))


def _pack2(v):
    return jnp.concatenate([v[:, :CE], v[:, :CE]], axis=1)


def kernel(images, stem_w, stem_s, stem_b,
           b1_exp_w, b1_exp_s, b1_exp_b, b1_dw_w, b1_dw_s, b1_dw_b,
           b1_proj_w, b1_proj_s, b1_proj_b,
           b2_exp_w, b2_exp_s, b2_exp_b, b2_dw_w, b2_dw_s, b2_dw_b,
           b2_proj_w, b2_proj_s, b2_proj_b,
           b3_exp_w, b3_exp_s, b3_exp_b, b3_dw_w, b3_dw_s, b3_dw_b,
           b3_proj_w, b3_proj_s, b3_proj_b,
           head_w, head_s, head_b):
    n = images.shape[0]
    f32 = jnp.float32
    pcat = _build_patches(images)

    sw = stem_w[:KP, :16]
    ss, sb = stem_s[:, :16], stem_b[:, :16]
    e1w = b1_exp_w[:16, :CE]
    e1sp, e1bp = _pack2(b1_exp_s), _pack2(b1_exp_b)
    d1wp = _pack2(b1_dw_w)
    d1sp, d1bp = _pack2(b1_dw_s), _pack2(b1_dw_b)
    z64_16 = jnp.zeros((CE, 16), jnp.bfloat16)
    p1lo = jnp.concatenate([b1_proj_w[:CE, :16], z64_16], axis=0)
    p1hi = jnp.concatenate([z64_16, b1_proj_w[:CE, :16]], axis=0)
    p1s, p1b = b1_proj_s[:, :16], b1_proj_b[:, :16]
    e2w = b2_exp_w[:16, :CE]
    e2sp, e2bp = _pack2(b2_exp_s), _pack2(b2_exp_b)
    d2wp = _pack2(b2_dw_w)
    d2sp, d2bp = _pack2(b2_dw_s), _pack2(b2_dw_b)
    p2z = jnp.concatenate(
        [b2_proj_w[:CE, :24], jnp.zeros((CE, 24), jnp.bfloat16)], axis=0)
    p2s, p2b = b2_proj_s[:, :24], b2_proj_b[:, :24]
    e3w = b3_exp_w[:24, :C3]
    e3s, e3b = b3_exp_s[:, :C3], b3_exp_b[:, :C3]
    d3w = b3_dw_w[:, :C3]
    d3s, d3b = b3_dw_s[:, :C3], b3_dw_b[:, :C3]
    p3w = b3_proj_w[:C3, :24]
    p3s, p3b = b3_proj_s[:, :24], b3_proj_b[:, :24]
    hw = head_w[:24, :]

    full = lambda i: (0, 0)
    ws = lambda r, c: pl.BlockSpec((r, c), full)

    out = pl.pallas_call(
        _body,
        grid=(n,),
        in_specs=[pl.BlockSpec((None, M1, KP), lambda i: (i, 0, 0)),
                  ws(KP, 16), ws(1, 16), ws(1, 16),
                  ws(16, CE), ws(1, C), ws(1, C),
                  ws(9, C), ws(1, C), ws(1, C),
                  ws(C, 16), ws(C, 16), ws(1, 16), ws(1, 16),
                  ws(16, CE), ws(1, C), ws(1, C),
                  ws(9, C), ws(1, C), ws(1, C),
                  ws(C, 24), ws(1, 24), ws(1, 24),
                  ws(24, C3), ws(1, C3), ws(1, C3),
                  ws(9, C3), ws(1, C3), ws(1, C3),
                  ws(C3, 24), ws(1, 24), ws(1, 24),
                  ws(24, C), ws(1, C), ws(1, C)],
        out_specs=pl.BlockSpec((None, 1, C), lambda i: (i, 0, 0)),
        out_shape=jax.ShapeDtypeStruct((n, 1, C), jnp.bfloat16),
        scratch_shapes=[
            pltpu.VMEM((H1 + 2, WH, C), jnp.float32),
            pltpu.VMEM((H1 + 2, WH + 2, C), jnp.float32),
            pltpu.VMEM((H2 + 2, H2 + 2, C3), jnp.float32),
        ],
        compiler_params=pltpu.CompilerParams(
            dimension_semantics=("parallel",)),
    )(pcat, sw, ss.astype(f32), sb.astype(f32),
      e1w, e1sp, e1bp, d1wp, d1sp, d1bp, p1lo, p1hi, p1s, p1b,
      e2w, e2sp, e2bp, d2wp, d2sp, d2bp, p2z, p2s, p2b,
      e3w, e3s, e3b, d3w, d3s, d3b, p3w, p3s, p3b,
      hw, head_s, head_b)
    return out.astype(images.dtype)
